# Initial kernel scaffold; baseline (speedup 1.0000x reference)
import jax
import jax.numpy as jnp
from jax.experimental import pallas as pl


def kernel(x, edge_index, batch, W1, b1, W2, b2, W3, b3, Wh, bh, Wo, bo):
    raise NotImplementedError("write your pallas kernel here")


# speedup vs baseline: 16.6200x; 16.6200x over previous
import functools

import jax
import jax.numpy as jnp
from jax import lax
from jax.experimental import pallas as pl
from jax.experimental.pallas import tpu as pltpu
from jax.experimental.pallas import tpu_sc as plsc

NC = 2
NS = 16
NW = NC * NS
C = 128


def _sc_edge_scatter(g, src_l, dst_l, n_pad, width, n_chunks):
    rps = n_pad // NS
    mesh = plsc.VectorSubcoreMesh(core_axis_name="c", subcore_axis_name="s")

    @functools.partial(
        pl.kernel,
        mesh=mesh,
        compiler_params=pltpu.CompilerParams(use_tc_tiling_on_sc=False),
        out_type=jax.ShapeDtypeStruct((NC, n_pad, width), jnp.float32),
        scratch_types=[
            pltpu.VMEM((C,), jnp.int32),
            pltpu.VMEM((C,), jnp.int32),
            pltpu.VMEM((C, width), jnp.float32),
            pltpu.VMEM_SHARED((n_pad, width), jnp.float32),
            pltpu.SemaphoreType.DMA,
        ],
    )
    def k(g_h, src_h, dst_h, out_h, sidx, didx, rows, acc, sem):
        cid = lax.axis_index("c")
        sid = lax.axis_index("s")
        sl = pl.ds(sid * rps, rps)
        pltpu.sync_copy(g_h.at[sl], acc.at[sl])
        plsc.subcore_barrier()
        w = cid * NS + sid

        def body(j, carry):
            pltpu.sync_copy(src_h.at[w, j], sidx)
            pltpu.sync_copy(dst_h.at[w, j], didx)
            pltpu.async_copy(g_h.at[sidx], rows, sem).wait()
            pltpu.sync_copy(rows, acc.at[didx], add=True)
            return carry

        lax.fori_loop(0, n_chunks, body, 0)
        plsc.subcore_barrier()
        pltpu.sync_copy(acc.at[sl], out_h.at[cid, sl])

    return k(g, src_l, dst_l)


def _sc_degree(dst_l, n_pad, n_chunks):
    rps = n_pad // NS
    assert rps % C == 0
    mesh = plsc.VectorSubcoreMesh(core_axis_name="c", subcore_axis_name="s")

    @functools.partial(
        pl.kernel,
        mesh=mesh,
        compiler_params=pltpu.CompilerParams(use_tc_tiling_on_sc=False),
        out_type=jax.ShapeDtypeStruct((NC, n_pad), jnp.float32),
        scratch_types=[
            pltpu.VMEM((C,), jnp.int32),
            pltpu.VMEM((C,), jnp.float32),
            pltpu.VMEM((C,), jnp.float32),
            pltpu.VMEM_SHARED((n_pad,), jnp.float32),
        ],
    )
    def k(dst_h, out_h, didx, ones_v, zero_v, acc):
        cid = lax.axis_index("c")
        sid = lax.axis_index("s")
        for i in range(C // 16):
            ones_v[pl.ds(i * 16, 16)] = jnp.ones((16,), jnp.float32)
            zero_v[pl.ds(i * 16, 16)] = jnp.zeros((16,), jnp.float32)
        for t in range(rps // C):
            pltpu.sync_copy(zero_v, acc.at[pl.ds(sid * rps + t * C, C)])
        plsc.subcore_barrier()
        w = cid * NS + sid

        def body(j, carry):
            pltpu.sync_copy(dst_h.at[w, j], didx)
            pltpu.sync_copy(ones_v, acc.at[didx], add=True)
            return carry

        lax.fori_loop(0, n_chunks, body, 0)
        plsc.subcore_barrier()
        pltpu.sync_copy(acc.at[pl.ds(sid * rps, rps)],
                        out_h.at[cid, pl.ds(sid * rps, rps)])

    return k(dst_l)


def _dinv(degT_ref):
    return lax.rsqrt(degT_ref[:, 0] + degT_ref[:, 1] + 1.0)[:, None]


def kernel(x, edge_index, batch, W1, b1, W2, b2, W3, b3, Wh, bh, Wo, bo):
    N, D = x.shape
    E = edge_index.shape[1]
    H = W1.shape[1]
    H2 = W2.shape[1]
    H3 = W3.shape[1]
    OUT = Wo.shape[1]

    n_pad = ((N + C - 1) // C + 1) * C
    n_pad = ((n_pad + NS * C - 1) // (NS * C)) * (NS * C)
    pad_rows = n_pad - N

    ew = ((E + NW * C - 1) // (NW * C)) * C
    n_chunks = ew // C
    e_pad = ew * NW

    src = edge_index[0]
    dst = edge_index[1]
    pi = jnp.arange(e_pad - E, dtype=jnp.int32)
    src_p = jnp.concatenate([src, N + (pi % pad_rows)])
    dst_p = jnp.concatenate([dst, N + ((pi * 7 + pad_rows // 2) % pad_rows)])
    src_l = src_p.reshape(NW, n_chunks, C)
    dst_l = dst_p.reshape(NW, n_chunks, C)
    x_pad = jnp.pad(x, ((0, pad_rows), (0, 0)))
    batch2d = jnp.pad(batch, (0, pad_rows), constant_values=64)[None, :]
    b1r, b2r, b3r = b1[None, :], b2[None, :], b3[None, :]
    bhr, bor = bh[None, :], bo[None, :]

    R = n_pad // NS
    T = NS

    deg_parts = _sc_degree(dst_l, n_pad, n_chunks)
    degT = deg_parts.T

    def _b_body(x_ref, w_ref, degT_ref, o_ref):
        h = jnp.dot(x_ref[...], w_ref[...], preferred_element_type=jnp.float32)
        o_ref[...] = h * _dinv(degT_ref)

    g1 = pl.pallas_call(
        _b_body,
        grid=(T,),
        in_specs=[pl.BlockSpec((R, D), lambda i: (i, 0)),
                  pl.BlockSpec((D, H), lambda i: (0, 0)),
                  pl.BlockSpec((R, 2), lambda i: (i, 0))],
        out_specs=pl.BlockSpec((R, H), lambda i: (i, 0)),
        out_shape=jax.ShapeDtypeStruct((n_pad, H), jnp.float32),
    )(x_pad, W1, degT)

    p1_ = _sc_edge_scatter(g1, src_l, dst_l, n_pad, H, n_chunks)

    def _c_body(p_ref, g_ref, degT_ref, b_ref, o_ref):
        dinv = _dinv(degT_ref)
        agg = dinv * (p_ref[0] + p_ref[1] - g_ref[...]) + b_ref[...]
        o_ref[...] = dinv * jnp.maximum(agg, 0.0)

    g2 = pl.pallas_call(
        _c_body,
        grid=(T,),
        in_specs=[pl.BlockSpec((NC, R, H), lambda i: (0, i, 0)),
                  pl.BlockSpec((R, H), lambda i: (i, 0)),
                  pl.BlockSpec((R, 2), lambda i: (i, 0)),
                  pl.BlockSpec((1, H), lambda i: (0, 0))],
        out_specs=pl.BlockSpec((R, H), lambda i: (i, 0)),
        out_shape=jax.ShapeDtypeStruct((n_pad, H), jnp.float32),
    )(p1_, g1, degT, b1r)

    p2_ = _sc_edge_scatter(g2, src_l, dst_l, n_pad, H, n_chunks)

    def _d_body(p_ref, g_ref, degT_ref, w_ref, b_ref, o_ref):
        dinv = _dinv(degT_ref)
        a = dinv * (p_ref[0] + p_ref[1] - g_ref[...])
        c2 = jnp.maximum(
            jnp.dot(a, w_ref[...], preferred_element_type=jnp.float32)
            + b_ref[...], 0.0)
        o_ref[...] = dinv * c2

    g3 = pl.pallas_call(
        _d_body,
        grid=(T,),
        in_specs=[pl.BlockSpec((NC, R, H), lambda i: (0, i, 0)),
                  pl.BlockSpec((R, H), lambda i: (i, 0)),
                  pl.BlockSpec((R, 2), lambda i: (i, 0)),
                  pl.BlockSpec((H, H2), lambda i: (0, 0)),
                  pl.BlockSpec((1, H2), lambda i: (0, 0))],
        out_specs=pl.BlockSpec((R, H2), lambda i: (i, 0)),
        out_shape=jax.ShapeDtypeStruct((n_pad, H2), jnp.float32),
    )(p2_, g2, degT, W2, b2r)

    p3_ = _sc_edge_scatter(g3, src_l, dst_l, n_pad, H2, n_chunks)

    GR = 64

    def _e_body(p_ref, g_ref, degT_ref, w_ref, b_ref, batch_ref,
                sums_ref, cnts_ref):
        i = pl.program_id(0)

        @pl.when(i == 0)
        def _():
            sums_ref[...] = jnp.zeros_like(sums_ref)
            cnts_ref[...] = jnp.zeros_like(cnts_ref)

        dinv = _dinv(degT_ref)
        a = dinv * (p_ref[0] + p_ref[1] - g_ref[...])
        c3 = jnp.maximum(
            jnp.dot(a, w_ref[...], preferred_element_type=jnp.float32)
            + b_ref[...], 0.0)
        onehot = (lax.broadcasted_iota(jnp.int32, (GR, R), 0)
                  == batch_ref[...]).astype(jnp.float32)
        sums_ref[...] += jnp.dot(onehot, c3,
                                 preferred_element_type=jnp.float32)
        cnts_ref[...] = cnts_ref[...] + jnp.sum(onehot, axis=1, keepdims=True)

    sums, cnts = pl.pallas_call(
        _e_body,
        grid=(T,),
        in_specs=[pl.BlockSpec((NC, R, H2), lambda i: (0, i, 0)),
                  pl.BlockSpec((R, H2), lambda i: (i, 0)),
                  pl.BlockSpec((R, 2), lambda i: (i, 0)),
                  pl.BlockSpec((H2, H3), lambda i: (0, 0)),
                  pl.BlockSpec((1, H3), lambda i: (0, 0)),
                  pl.BlockSpec((1, R), lambda i: (0, i))],
        out_specs=[pl.BlockSpec((GR, H3), lambda i: (0, 0)),
                   pl.BlockSpec((GR, 128), lambda i: (0, 0))],
        out_shape=[jax.ShapeDtypeStruct((GR, H3), jnp.float32),
                   jax.ShapeDtypeStruct((GR, 128), jnp.float32)],
    )(p3_, g3, degT, W3, b3r, batch2d)

    def _f_body(sums_ref, cnts_ref, wh_ref, bh_ref, wo_ref, bo_ref, o_ref):
        cnt = cnts_ref[:, 0:1]
        pooled = sums_ref[...] / jnp.maximum(cnt, 1.0)
        hid = jnp.maximum(
            jnp.dot(pooled, wh_ref[...], preferred_element_type=jnp.float32)
            + bh_ref[...], 0.0)
        logits = jnp.dot(hid, wo_ref[...],
                         preferred_element_type=jnp.float32) + bo_ref[...]
        o_ref[...] = jax.nn.sigmoid(logits)

    out = pl.pallas_call(
        _f_body,
        out_shape=jax.ShapeDtypeStruct((GR, OUT), jnp.float32),
    )(sums, cnts, Wh, bhr, Wo, bor)

    return out


# speedup vs baseline: 31.3209x; 1.8845x over previous
import functools

import jax
import jax.numpy as jnp
from jax import lax
from jax.experimental import pallas as pl
from jax.experimental.pallas import tpu as pltpu
from jax.experimental.pallas import tpu_sc as plsc

NC = 2
NS = 16
NW = NC * NS
C = 128


NBUF = 4


def _sc_edge_scatter(g, src_l, dst_l, n_pad, width, n_chunks):
    rps = n_pad // NS
    assert n_chunks % NBUF == 0
    mesh = plsc.VectorSubcoreMesh(core_axis_name="c", subcore_axis_name="s")

    @functools.partial(
        pl.kernel,
        mesh=mesh,
        compiler_params=pltpu.CompilerParams(use_tc_tiling_on_sc=False),
        out_type=jax.ShapeDtypeStruct((NC, n_pad, width), jnp.float32),
        scratch_types=[
            pltpu.VMEM((n_chunks, C), jnp.int32),
            pltpu.VMEM((n_chunks, C), jnp.int32),
        ]
        + [pltpu.VMEM((C, width), jnp.float32) for _ in range(NBUF)]
        + [
            pltpu.VMEM_SHARED((n_pad, width), jnp.float32),
        ]
        + [pltpu.SemaphoreType.DMA for _ in range(2 * NBUF)],
    )
    def k(g_h, src_h, dst_h, out_h, sall, dall, *rest):
        rows = rest[:NBUF]
        acc = rest[NBUF]
        gsem = rest[NBUF + 1:NBUF + 1 + NBUF]
        ssem = rest[NBUF + 1 + NBUF:]
        cid = lax.axis_index("c")
        sid = lax.axis_index("s")
        sl = pl.ds(sid * rps, rps)
        w = cid * NS + sid
        pltpu.sync_copy(src_h.at[w], sall)
        pltpu.sync_copy(dst_h.at[w], dall)
        pltpu.sync_copy(g_h.at[sl], acc.at[sl])
        plsc.subcore_barrier()

        def gath(j, b):
            return pltpu.async_copy(g_h.at[sall.at[j]], rows[b], gsem[b])

        def scat(j, b):
            return pltpu.async_copy(rows[b], acc.at[dall.at[j]], ssem[b],
                                    add=True)

        gath(0, 0)
        gath(1, 1)

        def body(i, carry):
            j0 = i * NBUF
            for u in range(NBUF):
                j = j0 + u
                b = u
                bn = (u + 2) % NBUF
                pltpu.make_async_copy(g_h.at[sall.at[j]], rows[b],
                                      gsem[b]).wait()
                scat(j, b)
                jp = j + 2 - NBUF
                @pl.when(jp >= 0)
                def _():
                    pltpu.make_async_copy(rows[bn], acc.at[dall.at[jp]],
                                          ssem[bn]).wait()
                @pl.when(j + 2 < n_chunks)
                def _():
                    gath(j + 2, bn)
            return carry

        lax.fori_loop(0, n_chunks // NBUF, body, 0, unroll=False)
        for j in (n_chunks - 2, n_chunks - 1):
            b = j % NBUF
            pltpu.make_async_copy(rows[b], acc.at[dall.at[j]],
                                  ssem[b]).wait()
        plsc.subcore_barrier()
        pltpu.sync_copy(acc.at[sl], out_h.at[cid, sl])

    return k(g, src_l, dst_l)


def _sc_degree(dst_l, n_pad, n_chunks):
    rps = n_pad // NS
    assert rps % C == 0 and n_chunks % NBUF == 0
    mesh = plsc.VectorSubcoreMesh(core_axis_name="c", subcore_axis_name="s")

    @functools.partial(
        pl.kernel,
        mesh=mesh,
        compiler_params=pltpu.CompilerParams(use_tc_tiling_on_sc=False),
        out_type=jax.ShapeDtypeStruct((NC, n_pad), jnp.float32),
        scratch_types=[
            pltpu.VMEM((n_chunks, C), jnp.int32),
            pltpu.VMEM((C,), jnp.float32),
            pltpu.VMEM((C,), jnp.float32),
            pltpu.VMEM_SHARED((n_pad,), jnp.float32),
        ]
        + [pltpu.SemaphoreType.DMA for _ in range(NBUF)],
    )
    def k(dst_h, out_h, dall, ones_v, zero_v, acc, *sems):
        cid = lax.axis_index("c")
        sid = lax.axis_index("s")
        for i in range(C // 16):
            ones_v[pl.ds(i * 16, 16)] = jnp.ones((16,), jnp.float32)
            zero_v[pl.ds(i * 16, 16)] = jnp.zeros((16,), jnp.float32)
        w = cid * NS + sid
        pltpu.sync_copy(dst_h.at[w], dall)
        for t in range(rps // C):
            pltpu.sync_copy(zero_v, acc.at[pl.ds(sid * rps + t * C, C)])
        plsc.subcore_barrier()

        def body(i, carry):
            for u in range(NBUF):
                j = i * NBUF + u
                @pl.when(i > 0)
                def _():
                    pltpu.make_async_copy(ones_v, acc.at[dall.at[j - NBUF]],
                                          sems[u]).wait()
                pltpu.async_copy(ones_v, acc.at[dall.at[j]], sems[u],
                                 add=True)
            return carry

        lax.fori_loop(0, n_chunks // NBUF, body, 0, unroll=False)
        for u in range(NBUF):
            j = n_chunks - NBUF + u
            pltpu.make_async_copy(ones_v, acc.at[dall.at[j]], sems[u]).wait()
        plsc.subcore_barrier()
        pltpu.sync_copy(acc.at[pl.ds(sid * rps, rps)],
                        out_h.at[cid, pl.ds(sid * rps, rps)])

    return k(dst_l)


def _dinv(degT_ref):
    return lax.rsqrt(degT_ref[:, 0] + degT_ref[:, 1] + 1.0)[:, None]


def kernel(x, edge_index, batch, W1, b1, W2, b2, W3, b3, Wh, bh, Wo, bo):
    N, D = x.shape
    E = edge_index.shape[1]
    H = W1.shape[1]
    H2 = W2.shape[1]
    H3 = W3.shape[1]
    OUT = Wo.shape[1]

    n_pad = ((N + C - 1) // C + 1) * C
    n_pad = ((n_pad + NS * C - 1) // (NS * C)) * (NS * C)
    pad_rows = n_pad - N

    bc = C * NBUF
    ew = ((E + NW * bc - 1) // (NW * bc)) * bc
    n_chunks = ew // C
    e_pad = ew * NW

    src = edge_index[0]
    dst = edge_index[1]
    pi = jnp.arange(e_pad - E, dtype=jnp.int32)
    src_p = jnp.concatenate([src, N + (pi % pad_rows)])
    dst_p = jnp.concatenate([dst, N + ((pi * 7 + pad_rows // 2) % pad_rows)])
    src_l = src_p.reshape(NW, n_chunks, C)
    dst_l = dst_p.reshape(NW, n_chunks, C)
    x_pad = jnp.pad(x, ((0, pad_rows), (0, 0)))
    batch2d = jnp.pad(batch, (0, pad_rows), constant_values=64)[None, :]
    b1r, b2r, b3r = b1[None, :], b2[None, :], b3[None, :]
    bhr, bor = bh[None, :], bo[None, :]

    R = n_pad // NS
    T = NS

    deg_parts = _sc_degree(dst_l, n_pad, n_chunks)
    degT = deg_parts.T

    def _b_body(x_ref, w_ref, degT_ref, o_ref):
        h = jnp.dot(x_ref[...], w_ref[...], preferred_element_type=jnp.float32)
        o_ref[...] = h * _dinv(degT_ref)

    g1 = pl.pallas_call(
        _b_body,
        grid=(T,),
        in_specs=[pl.BlockSpec((R, D), lambda i: (i, 0)),
                  pl.BlockSpec((D, H), lambda i: (0, 0)),
                  pl.BlockSpec((R, 2), lambda i: (i, 0))],
        out_specs=pl.BlockSpec((R, H), lambda i: (i, 0)),
        out_shape=jax.ShapeDtypeStruct((n_pad, H), jnp.float32),
    )(x_pad, W1, degT)

    p1_ = _sc_edge_scatter(g1, src_l, dst_l, n_pad, H, n_chunks)

    def _c_body(p_ref, g_ref, degT_ref, b_ref, o_ref):
        dinv = _dinv(degT_ref)
        agg = dinv * (p_ref[0] + p_ref[1] - g_ref[...]) + b_ref[...]
        o_ref[...] = dinv * jnp.maximum(agg, 0.0)

    g2 = pl.pallas_call(
        _c_body,
        grid=(T,),
        in_specs=[pl.BlockSpec((NC, R, H), lambda i: (0, i, 0)),
                  pl.BlockSpec((R, H), lambda i: (i, 0)),
                  pl.BlockSpec((R, 2), lambda i: (i, 0)),
                  pl.BlockSpec((1, H), lambda i: (0, 0))],
        out_specs=pl.BlockSpec((R, H), lambda i: (i, 0)),
        out_shape=jax.ShapeDtypeStruct((n_pad, H), jnp.float32),
    )(p1_, g1, degT, b1r)

    p2_ = _sc_edge_scatter(g2, src_l, dst_l, n_pad, H, n_chunks)

    def _d_body(p_ref, g_ref, degT_ref, w_ref, b_ref, o_ref):
        dinv = _dinv(degT_ref)
        a = dinv * (p_ref[0] + p_ref[1] - g_ref[...])
        c2 = jnp.maximum(
            jnp.dot(a, w_ref[...], preferred_element_type=jnp.float32)
            + b_ref[...], 0.0)
        o_ref[...] = dinv * c2

    g3 = pl.pallas_call(
        _d_body,
        grid=(T,),
        in_specs=[pl.BlockSpec((NC, R, H), lambda i: (0, i, 0)),
                  pl.BlockSpec((R, H), lambda i: (i, 0)),
                  pl.BlockSpec((R, 2), lambda i: (i, 0)),
                  pl.BlockSpec((H, H2), lambda i: (0, 0)),
                  pl.BlockSpec((1, H2), lambda i: (0, 0))],
        out_specs=pl.BlockSpec((R, H2), lambda i: (i, 0)),
        out_shape=jax.ShapeDtypeStruct((n_pad, H2), jnp.float32),
    )(p2_, g2, degT, W2, b2r)

    g3a = g3[:, :H]
    g3b = g3[:, H:]
    p3a = _sc_edge_scatter(g3a, src_l, dst_l, n_pad, H, n_chunks)
    p3b = _sc_edge_scatter(g3b, src_l, dst_l, n_pad, H, n_chunks)

    GR = 64

    def _e_body(pa_ref, pb_ref, ga_ref, gb_ref, degT_ref, w_ref, b_ref,
                batch_ref, sums_ref, cnts_ref):
        i = pl.program_id(0)

        @pl.when(i == 0)
        def _():
            sums_ref[...] = jnp.zeros_like(sums_ref)
            cnts_ref[...] = jnp.zeros_like(cnts_ref)

        dinv = _dinv(degT_ref)
        a = jnp.concatenate(
            [dinv * (pa_ref[0] + pa_ref[1] - ga_ref[...]),
             dinv * (pb_ref[0] + pb_ref[1] - gb_ref[...])], axis=1)
        c3 = jnp.maximum(
            jnp.dot(a, w_ref[...], preferred_element_type=jnp.float32)
            + b_ref[...], 0.0)
        onehot = (lax.broadcasted_iota(jnp.int32, (GR, R), 0)
                  == batch_ref[...]).astype(jnp.float32)
        sums_ref[...] += jnp.dot(onehot, c3,
                                 preferred_element_type=jnp.float32)
        cnts_ref[...] = cnts_ref[...] + jnp.sum(onehot, axis=1, keepdims=True)

    sums, cnts = pl.pallas_call(
        _e_body,
        grid=(T,),
        in_specs=[pl.BlockSpec((NC, R, H), lambda i: (0, i, 0)),
                  pl.BlockSpec((NC, R, H), lambda i: (0, i, 0)),
                  pl.BlockSpec((R, H), lambda i: (i, 0)),
                  pl.BlockSpec((R, H), lambda i: (i, 0)),
                  pl.BlockSpec((R, 2), lambda i: (i, 0)),
                  pl.BlockSpec((H2, H3), lambda i: (0, 0)),
                  pl.BlockSpec((1, H3), lambda i: (0, 0)),
                  pl.BlockSpec((1, R), lambda i: (0, i))],
        out_specs=[pl.BlockSpec((GR, H3), lambda i: (0, 0)),
                   pl.BlockSpec((GR, 128), lambda i: (0, 0))],
        out_shape=[jax.ShapeDtypeStruct((GR, H3), jnp.float32),
                   jax.ShapeDtypeStruct((GR, 128), jnp.float32)],
    )(p3a, p3b, g3a, g3b, degT, W3, b3r, batch2d)

    def _f_body(sums_ref, cnts_ref, wh_ref, bh_ref, wo_ref, bo_ref, o_ref):
        cnt = cnts_ref[:, 0:1]
        pooled = sums_ref[...] / jnp.maximum(cnt, 1.0)
        hid = jnp.maximum(
            jnp.dot(pooled, wh_ref[...], preferred_element_type=jnp.float32)
            + bh_ref[...], 0.0)
        logits = jnp.dot(hid, wo_ref[...],
                         preferred_element_type=jnp.float32) + bo_ref[...]
        o_ref[...] = jax.nn.sigmoid(logits)

    out = pl.pallas_call(
        _f_body,
        out_shape=jax.ShapeDtypeStruct((GR, OUT), jnp.float32),
    )(sums, cnts, Wh, bhr, Wo, bor)

    return out


# speedup vs baseline: 31.7281x; 1.0130x over previous
import functools

import jax
import jax.numpy as jnp
from jax import lax
from jax.experimental import pallas as pl
from jax.experimental.pallas import tpu as pltpu
from jax.experimental.pallas import tpu_sc as plsc

NC = 2
NS = 16
NW = NC * NS
C = 125


NBUF = 4


def _sc_edge_scatter(gs, src_l, dst_l, n_pad, width, n_chunks):
    rps = n_pad // NS
    ngs = len(gs)
    assert n_chunks % NBUF == 0
    mesh = plsc.VectorSubcoreMesh(core_axis_name="c", subcore_axis_name="s")

    @functools.partial(
        pl.kernel,
        mesh=mesh,
        compiler_params=pltpu.CompilerParams(use_tc_tiling_on_sc=False),
        out_type=tuple(
            jax.ShapeDtypeStruct((NC, n_pad, width), jnp.float32)
            for _ in range(ngs)),
        scratch_types=[
            pltpu.VMEM((n_chunks, C), jnp.int32),
            pltpu.VMEM((n_chunks, C), jnp.int32),
        ]
        + [pltpu.VMEM((C, width), jnp.float32) for _ in range(NBUF)]
        + [
            pltpu.VMEM_SHARED((n_pad, width), jnp.float32),
        ]
        + [pltpu.SemaphoreType.DMA for _ in range(2 * NBUF)],
    )
    def k(*args):
        g_hs = args[:ngs]
        src_h, dst_h = args[ngs], args[ngs + 1]
        out_hs = args[ngs + 2:2 * ngs + 2]
        sall, dall = args[2 * ngs + 2], args[2 * ngs + 3]
        rest = args[2 * ngs + 4:]
        rows = rest[:NBUF]
        acc = rest[NBUF]
        gsem = rest[NBUF + 1:NBUF + 1 + NBUF]
        ssem = rest[NBUF + 1 + NBUF:]
        cid = lax.axis_index("c")
        sid = lax.axis_index("s")
        sl = pl.ds(sid * rps, rps)
        w = cid * NS + sid
        pltpu.sync_copy(src_h.at[w], sall)
        pltpu.sync_copy(dst_h.at[w], dall)

        for g_h, out_h in zip(g_hs, out_hs):
            pltpu.sync_copy(g_h.at[sl], acc.at[sl])
            plsc.subcore_barrier()

            def gath(j, b):
                return pltpu.async_copy(g_h.at[sall.at[j]], rows[b],
                                        gsem[b])

            def scat(j, b):
                return pltpu.async_copy(rows[b], acc.at[dall.at[j]],
                                        ssem[b], add=True)

            gath(0, 0)
            gath(1, 1)

            def body(i, carry):
                j0 = i * NBUF
                for u in range(NBUF):
                    j = j0 + u
                    b = u
                    bn = (u + 2) % NBUF
                    pltpu.make_async_copy(g_h.at[sall.at[j]], rows[b],
                                          gsem[b]).wait()
                    scat(j, b)
                    jp = j + 2 - NBUF
                    @pl.when(jp >= 0)
                    def _():
                        pltpu.make_async_copy(rows[bn],
                                              acc.at[dall.at[jp]],
                                              ssem[bn]).wait()
                    @pl.when(j + 2 < n_chunks)
                    def _():
                        gath(j + 2, bn)
                return carry

            lax.fori_loop(0, n_chunks // NBUF, body, 0, unroll=False)
            for j in (n_chunks - 2, n_chunks - 1):
                b = j % NBUF
                pltpu.make_async_copy(rows[b], acc.at[dall.at[j]],
                                      ssem[b]).wait()
            plsc.subcore_barrier()
            pltpu.sync_copy(acc.at[sl], out_h.at[cid, sl])

    return k(*gs, src_l, dst_l)


def _sc_degree(dst_l, n_pad, n_chunks):
    rps = n_pad // NS
    ZB = 128
    assert rps % ZB == 0 and n_chunks % NBUF == 0
    mesh = plsc.VectorSubcoreMesh(core_axis_name="c", subcore_axis_name="s")

    @functools.partial(
        pl.kernel,
        mesh=mesh,
        compiler_params=pltpu.CompilerParams(use_tc_tiling_on_sc=False),
        out_type=jax.ShapeDtypeStruct((NC, n_pad), jnp.float32),
        scratch_types=[
            pltpu.VMEM((n_chunks, C), jnp.int32),
            pltpu.VMEM((ZB,), jnp.float32),
            pltpu.VMEM((ZB,), jnp.float32),
            pltpu.VMEM_SHARED((n_pad,), jnp.float32),
        ]
        + [pltpu.SemaphoreType.DMA for _ in range(NBUF)],
    )
    def k(dst_h, out_h, dall, ones_v, zero_v, acc, *sems):
        cid = lax.axis_index("c")
        sid = lax.axis_index("s")
        for i in range(ZB // 16):
            ones_v[pl.ds(i * 16, 16)] = jnp.ones((16,), jnp.float32)
            zero_v[pl.ds(i * 16, 16)] = jnp.zeros((16,), jnp.float32)
        w = cid * NS + sid
        pltpu.sync_copy(dst_h.at[w], dall)
        for t in range(rps // ZB):
            pltpu.sync_copy(zero_v, acc.at[pl.ds(sid * rps + t * ZB, ZB)])
        plsc.subcore_barrier()
        ones_c = ones_v.at[pl.ds(0, C)]

        def body(i, carry):
            for u in range(NBUF):
                j = i * NBUF + u
                @pl.when(i > 0)
                def _():
                    pltpu.make_async_copy(ones_c, acc.at[dall.at[j - NBUF]],
                                          sems[u]).wait()
                pltpu.async_copy(ones_c, acc.at[dall.at[j]], sems[u],
                                 add=True)
            return carry

        lax.fori_loop(0, n_chunks // NBUF, body, 0, unroll=False)
        for u in range(NBUF):
            j = n_chunks - NBUF + u
            pltpu.make_async_copy(ones_c, acc.at[dall.at[j]], sems[u]).wait()
        plsc.subcore_barrier()
        pltpu.sync_copy(acc.at[pl.ds(sid * rps, rps)],
                        out_h.at[cid, pl.ds(sid * rps, rps)])

    return k(dst_l)


def _dinv(degT_ref):
    return lax.rsqrt(degT_ref[:, 0] + degT_ref[:, 1] + 1.0)[:, None]


def kernel(x, edge_index, batch, W1, b1, W2, b2, W3, b3, Wh, bh, Wo, bo):
    N, D = x.shape
    E = edge_index.shape[1]
    H = W1.shape[1]
    H2 = W2.shape[1]
    H3 = W3.shape[1]
    OUT = Wo.shape[1]

    n_pad = ((N + 2047) // 2048) * 2048
    pad_rows = n_pad - N

    assert E % (NW * C * NBUF) == 0, "edge count must tile evenly"
    ew = E // NW
    n_chunks = ew // C

    src_l = edge_index[0].reshape(NW, n_chunks, C)
    dst_l = edge_index[1].reshape(NW, n_chunks, C)
    x_pad = jnp.pad(x, ((0, pad_rows), (0, 0)))
    batch2d = jnp.pad(batch, (0, pad_rows), constant_values=64)[None, :]
    b1r, b2r, b3r = b1[None, :], b2[None, :], b3[None, :]
    bhr, bor = bh[None, :], bo[None, :]

    R = n_pad // NS
    T = NS

    deg_parts = _sc_degree(dst_l, n_pad, n_chunks)
    degT = deg_parts.T

    def _b_body(x_ref, w_ref, degT_ref, o_ref):
        h = jnp.dot(x_ref[...], w_ref[...], preferred_element_type=jnp.float32)
        o_ref[...] = h * _dinv(degT_ref)

    g1 = pl.pallas_call(
        _b_body,
        grid=(T,),
        in_specs=[pl.BlockSpec((R, D), lambda i: (i, 0)),
                  pl.BlockSpec((D, H), lambda i: (0, 0)),
                  pl.BlockSpec((R, 2), lambda i: (i, 0))],
        out_specs=pl.BlockSpec((R, H), lambda i: (i, 0)),
        out_shape=jax.ShapeDtypeStruct((n_pad, H), jnp.float32),
    )(x_pad, W1, degT)

    (p1_,) = _sc_edge_scatter((g1,), src_l, dst_l, n_pad, H, n_chunks)

    def _c_body(p_ref, g_ref, degT_ref, b_ref, o_ref):
        dinv = _dinv(degT_ref)
        agg = dinv * (p_ref[0] + p_ref[1] - g_ref[...]) + b_ref[...]
        o_ref[...] = dinv * jnp.maximum(agg, 0.0)

    g2 = pl.pallas_call(
        _c_body,
        grid=(T,),
        in_specs=[pl.BlockSpec((NC, R, H), lambda i: (0, i, 0)),
                  pl.BlockSpec((R, H), lambda i: (i, 0)),
                  pl.BlockSpec((R, 2), lambda i: (i, 0)),
                  pl.BlockSpec((1, H), lambda i: (0, 0))],
        out_specs=pl.BlockSpec((R, H), lambda i: (i, 0)),
        out_shape=jax.ShapeDtypeStruct((n_pad, H), jnp.float32),
    )(p1_, g1, degT, b1r)

    (p2_,) = _sc_edge_scatter((g2,), src_l, dst_l, n_pad, H, n_chunks)

    def _d_body(p_ref, g_ref, degT_ref, w_ref, b_ref, oa_ref, ob_ref):
        dinv = _dinv(degT_ref)
        a = dinv * (p_ref[0] + p_ref[1] - g_ref[...])
        c2 = jnp.maximum(
            jnp.dot(a, w_ref[...], preferred_element_type=jnp.float32)
            + b_ref[...], 0.0)
        g3t = dinv * c2
        oa_ref[...] = g3t[:, :H]
        ob_ref[...] = g3t[:, H:]

    g3a, g3b = pl.pallas_call(
        _d_body,
        grid=(T,),
        in_specs=[pl.BlockSpec((NC, R, H), lambda i: (0, i, 0)),
                  pl.BlockSpec((R, H), lambda i: (i, 0)),
                  pl.BlockSpec((R, 2), lambda i: (i, 0)),
                  pl.BlockSpec((H, H2), lambda i: (0, 0)),
                  pl.BlockSpec((1, H2), lambda i: (0, 0))],
        out_specs=[pl.BlockSpec((R, H), lambda i: (i, 0)),
                   pl.BlockSpec((R, H), lambda i: (i, 0))],
        out_shape=[jax.ShapeDtypeStruct((n_pad, H), jnp.float32),
                   jax.ShapeDtypeStruct((n_pad, H), jnp.float32)],
    )(p2_, g2, degT, W2, b2r)

    p3a, p3b = _sc_edge_scatter((g3a, g3b), src_l, dst_l, n_pad, H,
                                n_chunks)

    GR = 64

    def _e_body(pa_ref, pb_ref, ga_ref, gb_ref, degT_ref, w_ref, b_ref,
                batch_ref, sums_ref, cnts_ref):
        i = pl.program_id(0)

        @pl.when(i == 0)
        def _():
            sums_ref[...] = jnp.zeros_like(sums_ref)
            cnts_ref[...] = jnp.zeros_like(cnts_ref)

        dinv = _dinv(degT_ref)
        a = jnp.concatenate(
            [dinv * (pa_ref[0] + pa_ref[1] - ga_ref[...]),
             dinv * (pb_ref[0] + pb_ref[1] - gb_ref[...])], axis=1)
        c3 = jnp.maximum(
            jnp.dot(a, w_ref[...], preferred_element_type=jnp.float32)
            + b_ref[...], 0.0)
        onehot = (lax.broadcasted_iota(jnp.int32, (GR, R), 0)
                  == batch_ref[...]).astype(jnp.float32)
        sums_ref[...] += jnp.dot(onehot, c3,
                                 preferred_element_type=jnp.float32)
        cnts_ref[...] = cnts_ref[...] + jnp.sum(onehot, axis=1, keepdims=True)

    sums, cnts = pl.pallas_call(
        _e_body,
        grid=(T,),
        in_specs=[pl.BlockSpec((NC, R, H), lambda i: (0, i, 0)),
                  pl.BlockSpec((NC, R, H), lambda i: (0, i, 0)),
                  pl.BlockSpec((R, H), lambda i: (i, 0)),
                  pl.BlockSpec((R, H), lambda i: (i, 0)),
                  pl.BlockSpec((R, 2), lambda i: (i, 0)),
                  pl.BlockSpec((H2, H3), lambda i: (0, 0)),
                  pl.BlockSpec((1, H3), lambda i: (0, 0)),
                  pl.BlockSpec((1, R), lambda i: (0, i))],
        out_specs=[pl.BlockSpec((GR, H3), lambda i: (0, 0)),
                   pl.BlockSpec((GR, 128), lambda i: (0, 0))],
        out_shape=[jax.ShapeDtypeStruct((GR, H3), jnp.float32),
                   jax.ShapeDtypeStruct((GR, 128), jnp.float32)],
    )(p3a, p3b, g3a, g3b, degT, W3, b3r, batch2d)

    def _f_body(sums_ref, cnts_ref, wh_ref, bh_ref, wo_ref, bo_ref, o_ref):
        cnt = cnts_ref[:, 0:1]
        pooled = sums_ref[...] / jnp.maximum(cnt, 1.0)
        hid = jnp.maximum(
            jnp.dot(pooled, wh_ref[...], preferred_element_type=jnp.float32)
            + bh_ref[...], 0.0)
        logits = jnp.dot(hid, wo_ref[...],
                         preferred_element_type=jnp.float32) + bo_ref[...]
        o_ref[...] = jax.nn.sigmoid(logits)

    out = pl.pallas_call(
        _f_body,
        out_shape=jax.ShapeDtypeStruct((GR, OUT), jnp.float32),
    )(sums, cnts, Wh, bhr, Wo, bor)

    return out
